# baseline (speedup 1.0000x reference)
import functools
import jax
import jax.numpy as jnp
from jax import lax
from jax.experimental import pallas as pl
from jax.experimental.pallas import tpu as pltpu
from jax.experimental.pallas import tpu_sc as plsc

N_ENT = 1000000
N_REL = 1000
DIM = 64
B = 16384

NC = 2
NS = 16
L = 16
NW = NC * NS
BPW = B // NW
CHUNK = 128
NCHUNK = BPW // CHUNK


def _sc_body(hidx, tidx, ridx, eh, rvh, b0, b1, out,
             hidx_v, tidx_v, ridx_v, h_v, t_v, r_v, b0_v, b1_v, out_v, sem):
    wid = lax.axis_index("s") * NC + lax.axis_index("c")
    base = wid * BPW

    pltpu.sync_copy(hidx.at[pl.ds(base, BPW)], hidx_v)
    pltpu.sync_copy(tidx.at[pl.ds(base, BPW)], tidx_v)
    pltpu.sync_copy(ridx.at[pl.ds(base, BPW)], ridx_v)

    copies = []
    for j in range(NCHUNK):
        s = pl.ds(j * CHUNK, CHUNK)
        copies.append(pltpu.async_copy(eh.at[hidx_v.at[s]], h_v.at[s], sem))
        copies.append(pltpu.async_copy(eh.at[tidx_v.at[s]], t_v.at[s], sem))
        copies.append(pltpu.async_copy(rvh.at[ridx_v.at[s]], r_v.at[s], sem))
        copies.append(pltpu.async_copy(b0.at[hidx_v.at[s]], b0_v.at[s], sem))
        copies.append(pltpu.async_copy(b1.at[tidx_v.at[s]], b1_v.at[s], sem))
    for c in copies:
        c.wait()

    lanes = lax.iota(jnp.int32, L)

    def group(g, carry):
        rows = g * L + lanes
        acc = jnp.zeros((L,), jnp.float32)
        for d in range(DIM):
            col = jnp.full((L,), d, jnp.int32)
            hv = plsc.load_gather(h_v, [rows, col])
            tv = plsc.load_gather(t_v, [rows, col])
            rv = plsc.load_gather(r_v, [rows, col])
            diff = hv - tv - rv
            acc = acc + diff * diff
        sl = pl.ds(g * L, L)
        out_v[sl] = b0_v[sl] + b1_v[sl] - acc
        return carry

    lax.fori_loop(0, BPW // L, group, 0, unroll=False)

    pltpu.sync_copy(out_v, out.at[pl.ds(base, BPW)])


@functools.partial(jax.jit, static_argnames=())
def kernel(head_idx, rel1_idx, tail_idx, rel2_idx, Eh, rvh, bias0, bias1):
    del rel2_idx
    mesh = plsc.VectorSubcoreMesh(core_axis_name="c", subcore_axis_name="s")
    f = pl.kernel(
        _sc_body,
        out_type=jax.ShapeDtypeStruct((B,), jnp.float32),
        mesh=mesh,
        scratch_types=[
            pltpu.VMEM((BPW,), jnp.int32),
            pltpu.VMEM((BPW,), jnp.int32),
            pltpu.VMEM((BPW,), jnp.int32),
            pltpu.VMEM((BPW, DIM), jnp.float32),
            pltpu.VMEM((BPW, DIM), jnp.float32),
            pltpu.VMEM((BPW, DIM), jnp.float32),
            pltpu.VMEM((BPW,), jnp.float32),
            pltpu.VMEM((BPW,), jnp.float32),
            pltpu.VMEM((BPW,), jnp.float32),
            pltpu.SemaphoreType.DMA,
        ],
        compiler_params=pltpu.CompilerParams(
            needs_layout_passes=False, use_tc_tiling_on_sc=False),
    )
    return f(head_idx.astype(jnp.int32), tail_idx.astype(jnp.int32),
             rel1_idx.astype(jnp.int32), Eh, rvh, bias0, bias1)


# speedup vs baseline: 1.0566x; 1.0566x over previous
import functools
import jax
import jax.numpy as jnp
from jax import lax
from jax.experimental import pallas as pl
from jax.experimental.pallas import tpu as pltpu
from jax.experimental.pallas import tpu_sc as plsc

N_ENT = 1000000
N_REL = 1000
DIM = 64
B = 16384

NC = 2
NS = 16
L = 16
NW = NC * NS
BPW = B // NW
R = 2 * B
BLK = 256
NBLK_FULL = 3906
TAIL_BASE = NBLK_FULL * BLK
IT_FULL = 122
DUMP = R
EPAD = 1 << 20
STAGE_ROWS = R + 128
SCB = 128


def _k1_body(hidx, tidx, ehT, eh_tail, stage,
             idx_all, hits, col0, col1, stage_b, slotbuf, tmp16,
             sem0, sem1, ssem):
    wid = lax.axis_index("s") * NC + lax.axis_index("c")
    lanes = lax.iota(jnp.int32, L)

    pltpu.sync_copy(hidx, idx_all.at[pl.ds(0, B)])
    pltpu.sync_copy(tidx, idx_all.at[pl.ds(B, B)])
    idx_all[pl.ds(R, L)] = jnp.full((L,), EPAD, jnp.int32)

    def scan(c, nh):
        e = idx_all[pl.ds(c * L, L)]
        slots = jnp.full((L,), c * L, jnp.int32) + lanes
        m = lax.bitwise_and(lax.shift_right_logical(e, 8),
                            jnp.full((L,), NW - 1, jnp.int32)) == wid
        plsc.store_compressed(hits.at[pl.ds(nh, L)], slots, mask=m)
        n = plsc.all_reduce_population_count(m)[0]
        return nh + n

    nhits = lax.fori_loop(0, R // L, scan, 0, unroll=False)
    hits[pl.ds(nhits, L)] = jnp.full((L,), DUMP, jnp.int32)
    nchunks = lax.shift_right_logical(nhits + (L - 1), 4)

    def prefill(i, carry):
        slotbuf[pl.ds(i * L, L)] = jnp.full((L,), DUMP, jnp.int32)
        return carry

    lax.fori_loop(0, SCB // L, prefill, 0, unroll=False)
    tmp16[pl.ds(L, L)] = jnp.full((L,), DUMP, jnp.int32)

    def flush(cnt, force):
        do = jnp.logical_or(cnt > SCB - L, jnp.logical_and(force, cnt > 0))

        @pl.when(do)
        def _():
            pltpu.async_copy(stage_b, stage.at[slotbuf], ssem).wait()
            lax.fori_loop(0, SCB // L, prefill, 0, unroll=False)

        return jnp.where(do, 0, cnt)

    def process_block(b, col, cnt):
        bvec = jnp.full((L,), 0, jnp.int32) + b

        def chunk(c, cnt2):
            slots = hits[pl.ds(c * L, L)]
            e = plsc.load_gather(idx_all, [slots])
            m = lax.shift_right_logical(e, 8) == bvec
            tmp16[pl.ds(0, L)] = jnp.full((L,), DUMP, jnp.int32)
            plsc.store_compressed(tmp16.at[pl.ds(0, L)], slots, mask=m)
            n = plsc.all_reduce_population_count(m)[0]

            def extract(j, cnt3):
                slot = tmp16[pl.ds(j, L)][0]
                ev = idx_all[pl.ds(slot, L)][0]
                el = lax.bitwise_and(ev, BLK - 1)
                elv = jnp.full((L,), 0, jnp.int32) + el
                rowv = jnp.full((L,), 0, jnp.int32) + cnt3
                for k in range(DIM // L):
                    dv = jnp.full((L,), k * L, jnp.int32) + lanes
                    v = plsc.load_gather(col, [dv, elv])
                    plsc.store_scatter(stage_b, [rowv, dv], v)
                plsc.store_scatter(
                    slotbuf, [rowv],
                    jnp.full((L,), 0, jnp.int32) + slot,
                    mask=lanes == 0)
                return cnt3 + 1

            cnt2 = lax.fori_loop(0, n, extract, cnt2, unroll=False)
            return flush(cnt2, False)

        return lax.fori_loop(0, nchunks, chunk, cnt, unroll=False)

    def pbase(i):
        return jnp.minimum((wid + NW * i) * BLK, (NBLK_FULL - 1) * BLK)

    pltpu.async_copy(ehT.at[:, pl.ds(pbase(0), BLK)], col0, sem0)
    pltpu.async_copy(ehT.at[:, pl.ds(pbase(1), BLK)], col1, sem1)

    def pair(p, cnt):
        pltpu.make_async_copy(ehT.at[:, pl.ds(pbase(2 * p), BLK)],
                              col0, sem0).wait()
        cnt = process_block(wid + NW * 2 * p, col0, cnt)
        pltpu.async_copy(ehT.at[:, pl.ds(pbase(2 * p + 2), BLK)], col0, sem0)
        pltpu.make_async_copy(ehT.at[:, pl.ds(pbase(2 * p + 1), BLK)],
                              col1, sem1).wait()
        cnt = process_block(wid + NW * (2 * p + 1), col1, cnt)
        pltpu.async_copy(ehT.at[:, pl.ds(pbase(2 * p + 3), BLK)], col1, sem1)
        return cnt

    cnt = lax.fori_loop(0, IT_FULL // 2, pair, 0, unroll=False)

    pltpu.make_async_copy(ehT.at[:, pl.ds(pbase(IT_FULL), BLK)],
                          col0, sem0).wait()
    pltpu.make_async_copy(ehT.at[:, pl.ds(pbase(IT_FULL + 1), BLK)],
                          col1, sem1).wait()

    @pl.when(wid < 2)
    def _():
        pltpu.sync_copy(ehT.at[:, pl.ds((wid + NW * IT_FULL) * BLK, BLK)],
                        col0)

    cnt = lax.cond(wid < 2,
                   lambda c: process_block(wid + NW * IT_FULL, col0, c),
                   lambda c: c, cnt)

    @pl.when(wid == 2)
    def _():
        pltpu.sync_copy(eh_tail, col0.at[:, pl.ds(0, 2 * DIM)])

    cnt = lax.cond(wid == 2,
                   lambda c: process_block(NBLK_FULL, col0, c),
                   lambda c: c, cnt)

    flush(cnt, True)


def _k2_body(stage, hidx, tidx, ridx, rvh2, b0, b1, out,
             rel_v, rpair_v, hi_v, ti_v, h_r, t_r, rv_r, b0_v, b1_v, out_v,
             sem):
    wid = lax.axis_index("s") * NC + lax.axis_index("c")
    base = wid * BPW
    lanes = lax.iota(jnp.int32, L)

    pltpu.sync_copy(ridx.at[pl.ds(base, BPW)], rel_v)
    pltpu.sync_copy(hidx.at[pl.ds(base, BPW)], hi_v)
    pltpu.sync_copy(tidx.at[pl.ds(base, BPW)], ti_v)

    def pairs(i, carry):
        s = pl.ds(i * L, L)
        rpair_v[s] = lax.shift_right_logical(rel_v[s], 1)
        return carry

    lax.fori_loop(0, BPW // L, pairs, 0, unroll=False)

    NBAT = BPW // SCB

    def batch(bi, carry):
        s0 = base + bi * SCB
        c1 = pltpu.async_copy(stage.at[pl.ds(s0, SCB), :], h_r, sem)
        c2 = pltpu.async_copy(stage.at[pl.ds(B + s0, SCB), :], t_r, sem)
        c3 = pltpu.async_copy(rvh2.at[rpair_v.at[pl.ds(bi * SCB, SCB)]],
                              rv_r, sem)
        c4 = pltpu.async_copy(b0.at[hi_v.at[pl.ds(bi * SCB, SCB)]], b0_v, sem)
        c5 = pltpu.async_copy(b1.at[ti_v.at[pl.ds(bi * SCB, SCB)]], b1_v, sem)
        c1.wait(); c2.wait(); c3.wait(); c4.wait(); c5.wait()

        def group(g, carry2):
            req = jnp.full((L,), g * L, jnp.int32) + lanes
            rh = lax.bitwise_and(
                rel_v[pl.ds(bi * SCB + g * L, L)],
                jnp.full((L,), 1, jnp.int32)) * DIM
            acc = jnp.zeros((L,), jnp.float32)
            for d in range(DIM):
                col = jnp.full((L,), d, jnp.int32)
                hv = plsc.load_gather(h_r, [req, col])
                tv = plsc.load_gather(t_r, [req, col])
                rv = plsc.load_gather(rv_r, [req, rh + col])
                diff = hv - tv - rv
                acc = acc + diff * diff
            gs = pl.ds(g * L, L)
            out_v[pl.ds(bi * SCB + g * L, L)] = b0_v[gs] + b1_v[gs] - acc
            return carry2

        lax.fori_loop(0, SCB // L, group, 0, unroll=False)
        return carry

    lax.fori_loop(0, NBAT, batch, 0, unroll=False)

    pltpu.sync_copy(out_v, out.at[pl.ds(base, BPW)])


@functools.partial(jax.jit, static_argnames=())
def kernel(head_idx, rel1_idx, tail_idx, rel2_idx, Eh, rvh, bias0, bias1):
    del rel2_idx
    hidx = head_idx.astype(jnp.int32)
    tidx = tail_idx.astype(jnp.int32)
    ridx = rel1_idx.astype(jnp.int32)
    ehT = Eh.T
    eh_tail = jnp.pad(Eh[TAIL_BASE:, :].T, ((0, 0), (0, DIM)))
    rvh2 = rvh.reshape(N_REL // 2, 2 * DIM)
    mesh = plsc.VectorSubcoreMesh(core_axis_name="c", subcore_axis_name="s")

    k1 = pl.kernel(
        _k1_body,
        out_type=jax.ShapeDtypeStruct((STAGE_ROWS, 2 * DIM), jnp.float32),
        mesh=mesh,
        scratch_types=[
            pltpu.VMEM((R + L,), jnp.int32),
            pltpu.VMEM((R + L,), jnp.int32),
            pltpu.VMEM((DIM, BLK), jnp.float32),
            pltpu.VMEM((DIM, BLK), jnp.float32),
            pltpu.VMEM((SCB, 2 * DIM), jnp.float32),
            pltpu.VMEM((SCB,), jnp.int32),
            pltpu.VMEM((2 * L,), jnp.int32),
            pltpu.SemaphoreType.DMA,
            pltpu.SemaphoreType.DMA,
            pltpu.SemaphoreType.DMA,
        ],
        compiler_params=pltpu.CompilerParams(needs_layout_passes=False),
    )
    stage = k1(hidx, tidx, ehT, eh_tail)

    k2 = pl.kernel(
        _k2_body,
        out_type=jax.ShapeDtypeStruct((B,), jnp.float32),
        mesh=mesh,
        scratch_types=[
            pltpu.VMEM((BPW,), jnp.int32),
            pltpu.VMEM((BPW,), jnp.int32),
            pltpu.VMEM((BPW,), jnp.int32),
            pltpu.VMEM((BPW,), jnp.int32),
            pltpu.VMEM((SCB, 2 * DIM), jnp.float32),
            pltpu.VMEM((SCB, 2 * DIM), jnp.float32),
            pltpu.VMEM((SCB, 2 * DIM), jnp.float32),
            pltpu.VMEM((SCB,), jnp.float32),
            pltpu.VMEM((SCB,), jnp.float32),
            pltpu.VMEM((BPW,), jnp.float32),
            pltpu.SemaphoreType.DMA,
        ],
        compiler_params=pltpu.CompilerParams(needs_layout_passes=False),
    )
    return k2(stage, hidx, tidx, ridx, rvh2, bias0, bias1)


# speedup vs baseline: 1.4674x; 1.3888x over previous
import functools
import jax
import jax.numpy as jnp
from jax import lax
from jax.experimental import pallas as pl
from jax.experimental.pallas import tpu as pltpu
from jax.experimental.pallas import tpu_sc as plsc

N_ENT = 1000000
N_REL = 1000
DIM = 64
B = 16384

NC = 2
NS = 16
L = 16
NW = NC * NS
BPW = B // NW
R = 2 * B
BLK = 256
NBLK_FULL = 3906
TAIL_BASE = NBLK_FULL * BLK
IT_FULL = 122
DUMP = R
EPAD = 1 << 20
STAGE_ROWS = R + 128
SCB = 128


def _k1_body(hidx, tidx, ehT, eh_tail, stage,
             idx_all, horder, col0, col1, stage_b, slotbuf, tmp16, tmp16e,
             cnt_v, off_v, cur_v,
             sem0, sem1, ssem):
    wid = lax.axis_index("s") * NC + lax.axis_index("c")
    lanes = lax.iota(jnp.int32, L)
    zero16 = jnp.full((L,), 0, jnp.int32)

    pltpu.sync_copy(hidx, idx_all.at[pl.ds(0, B)])
    pltpu.sync_copy(tidx, idx_all.at[pl.ds(B, B)])
    idx_all[pl.ds(R, L)] = jnp.full((L,), EPAD, jnp.int32)

    def store1(ref, pos, val):
        plsc.store_scatter(ref, [zero16 + pos], zero16 + val,
                           mask=lanes == 0)

    def read1(ref, pos):
        return ref[pl.ds(pos, L)][0]

    def zcnt(i, carry):
        cnt_v[pl.ds(i * L, L)] = zero16
        return carry

    lax.fori_loop(0, (IT_FULL + 2 + L) // L + 1, zcnt, 0, unroll=False)
    tmp16[pl.ds(L, L)] = jnp.full((L,), DUMP, jnp.int32)
    tmp16e[pl.ds(L, L)] = jnp.full((L,), EPAD, jnp.int32)

    def scan1(c, carry):
        e = idx_all[pl.ds(c * L, L)]
        m = lax.bitwise_and(lax.shift_right_logical(e, 8),
                            jnp.full((L,), NW - 1, jnp.int32)) == wid
        tmp16e[pl.ds(0, L)] = jnp.full((L,), EPAD, jnp.int32)
        plsc.store_compressed(tmp16e.at[pl.ds(0, L)], e, mask=m)
        n = plsc.all_reduce_population_count(m)[0]

        def count(j, carry2):
            ev = read1(tmp16e, j)
            o = lax.shift_right_logical(ev, 13)
            store1(cnt_v, o, read1(cnt_v, o) + 1)
            return carry2

        lax.fori_loop(0, n, count, 0, unroll=False)
        return carry

    lax.fori_loop(0, R // L, scan1, 0, unroll=False)

    def prefix(i, s):
        store1(off_v, i, s)
        store1(cur_v, i, s)
        return s + read1(cnt_v, i)

    total = lax.fori_loop(0, IT_FULL + 2, prefix, 0, unroll=False)
    store1(off_v, IT_FULL + 2, total)

    def scan2(c, carry):
        e = idx_all[pl.ds(c * L, L)]
        slots = jnp.full((L,), c * L, jnp.int32) + lanes
        m = lax.bitwise_and(lax.shift_right_logical(e, 8),
                            jnp.full((L,), NW - 1, jnp.int32)) == wid
        tmp16e[pl.ds(0, L)] = jnp.full((L,), EPAD, jnp.int32)
        tmp16[pl.ds(0, L)] = jnp.full((L,), DUMP, jnp.int32)
        plsc.store_compressed(tmp16e.at[pl.ds(0, L)], e, mask=m)
        plsc.store_compressed(tmp16.at[pl.ds(0, L)], slots, mask=m)
        n = plsc.all_reduce_population_count(m)[0]

        def place(j, carry2):
            ev = read1(tmp16e, j)
            slot = read1(tmp16, j)
            o = lax.shift_right_logical(ev, 13)
            p = read1(cur_v, o)
            store1(horder, p, slot)
            store1(cur_v, o, p + 1)
            return carry2

        lax.fori_loop(0, n, place, 0, unroll=False)
        return carry

    lax.fori_loop(0, R // L, scan2, 0, unroll=False)

    def prefill(i, carry):
        slotbuf[pl.ds(i * L, L)] = jnp.full((L,), DUMP, jnp.int32)
        return carry

    lax.fori_loop(0, SCB // L, prefill, 0, unroll=False)

    def process_block(i, col, cnt):
        lo = read1(off_v, i)
        hi = read1(off_v, i + 1)

        def extract(j, cnt3):
            flushing = cnt3 == SCB

            @pl.when(flushing)
            def _():
                pltpu.async_copy(stage_b, stage.at[slotbuf], ssem).wait()
                lax.fori_loop(0, SCB // L, prefill, 0, unroll=False)

            cnt3 = jnp.where(flushing, 0, cnt3)
            slot = read1(horder, j)
            ev = read1(idx_all, slot)
            el = lax.bitwise_and(ev, BLK - 1)
            elv = zero16 + el
            rowv = zero16 + cnt3
            for k in range(DIM // L):
                dv = jnp.full((L,), k * L, jnp.int32) + lanes
                v = plsc.load_gather(col, [dv, elv])
                plsc.store_scatter(stage_b, [rowv, dv], v)
            store1(slotbuf, cnt3, slot)
            return cnt3 + 1

        return lax.fori_loop(lo, hi, extract, cnt, unroll=False)

    def pbase(i):
        return jnp.minimum((wid + NW * i) * BLK, (NBLK_FULL - 1) * BLK)

    pltpu.async_copy(ehT.at[:, pl.ds(pbase(0), BLK)], col0, sem0)
    pltpu.async_copy(ehT.at[:, pl.ds(pbase(1), BLK)], col1, sem1)

    def pair(p, cnt):
        pltpu.make_async_copy(ehT.at[:, pl.ds(pbase(2 * p), BLK)],
                              col0, sem0).wait()
        cnt = process_block(2 * p, col0, cnt)
        pltpu.async_copy(ehT.at[:, pl.ds(pbase(2 * p + 2), BLK)], col0, sem0)
        pltpu.make_async_copy(ehT.at[:, pl.ds(pbase(2 * p + 1), BLK)],
                              col1, sem1).wait()
        cnt = process_block(2 * p + 1, col1, cnt)
        pltpu.async_copy(ehT.at[:, pl.ds(pbase(2 * p + 3), BLK)], col1, sem1)
        return cnt

    cnt = lax.fori_loop(0, IT_FULL // 2, pair, 0, unroll=False)

    pltpu.make_async_copy(ehT.at[:, pl.ds(pbase(IT_FULL), BLK)],
                          col0, sem0).wait()
    pltpu.make_async_copy(ehT.at[:, pl.ds(pbase(IT_FULL + 1), BLK)],
                          col1, sem1).wait()

    @pl.when(wid < 2)
    def _():
        pltpu.sync_copy(ehT.at[:, pl.ds((wid + NW * IT_FULL) * BLK, BLK)],
                        col0)

    @pl.when(wid == 2)
    def _():
        pltpu.sync_copy(eh_tail, col0.at[:, pl.ds(0, 2 * DIM)])

    cnt = process_block(IT_FULL, col0, cnt)

    @pl.when(cnt > 0)
    def _():
        pltpu.async_copy(stage_b, stage.at[slotbuf], ssem).wait()


def _k2_body(stage, hidx, tidx, ridx, rvh2, b0, b1, out,
             rel_v, rpair_v, hi_v, ti_v, h_r, t_r, rv_r, b0_v, b1_v, out_v,
             sem):
    wid = lax.axis_index("s") * NC + lax.axis_index("c")
    base = wid * BPW
    lanes = lax.iota(jnp.int32, L)

    pltpu.sync_copy(ridx.at[pl.ds(base, BPW)], rel_v)
    pltpu.sync_copy(hidx.at[pl.ds(base, BPW)], hi_v)
    pltpu.sync_copy(tidx.at[pl.ds(base, BPW)], ti_v)

    def pairs(i, carry):
        s = pl.ds(i * L, L)
        rpair_v[s] = lax.shift_right_logical(rel_v[s], 1)
        return carry

    lax.fori_loop(0, BPW // L, pairs, 0, unroll=False)

    NBAT = BPW // SCB

    def batch(bi, carry):
        s0 = base + bi * SCB
        c1 = pltpu.async_copy(stage.at[pl.ds(s0, SCB), :], h_r, sem)
        c2 = pltpu.async_copy(stage.at[pl.ds(B + s0, SCB), :], t_r, sem)
        c3 = pltpu.async_copy(rvh2.at[rpair_v.at[pl.ds(bi * SCB, SCB)]],
                              rv_r, sem)
        c4 = pltpu.async_copy(b0.at[hi_v.at[pl.ds(bi * SCB, SCB)]], b0_v, sem)
        c5 = pltpu.async_copy(b1.at[ti_v.at[pl.ds(bi * SCB, SCB)]], b1_v, sem)
        c1.wait(); c2.wait(); c3.wait(); c4.wait(); c5.wait()

        def group(g, carry2):
            req = jnp.full((L,), g * L, jnp.int32) + lanes
            rh = lax.bitwise_and(
                rel_v[pl.ds(bi * SCB + g * L, L)],
                jnp.full((L,), 1, jnp.int32)) * DIM
            acc = jnp.zeros((L,), jnp.float32)
            for d in range(DIM):
                col = jnp.full((L,), d, jnp.int32)
                hv = plsc.load_gather(h_r, [req, col])
                tv = plsc.load_gather(t_r, [req, col])
                rv = plsc.load_gather(rv_r, [req, rh + col])
                diff = hv - tv - rv
                acc = acc + diff * diff
            gs = pl.ds(g * L, L)
            out_v[pl.ds(bi * SCB + g * L, L)] = b0_v[gs] + b1_v[gs] - acc
            return carry2

        lax.fori_loop(0, SCB // L, group, 0, unroll=False)
        return carry

    lax.fori_loop(0, NBAT, batch, 0, unroll=False)

    pltpu.sync_copy(out_v, out.at[pl.ds(base, BPW)])


@functools.partial(jax.jit, static_argnames=())
def kernel(head_idx, rel1_idx, tail_idx, rel2_idx, Eh, rvh, bias0, bias1):
    del rel2_idx
    hidx = head_idx.astype(jnp.int32)
    tidx = tail_idx.astype(jnp.int32)
    ridx = rel1_idx.astype(jnp.int32)
    ehT = Eh.T
    eh_tail = jnp.pad(Eh[TAIL_BASE:, :].T, ((0, 0), (0, DIM)))
    rvh2 = rvh.reshape(N_REL // 2, 2 * DIM)
    mesh = plsc.VectorSubcoreMesh(core_axis_name="c", subcore_axis_name="s")

    k1 = pl.kernel(
        _k1_body,
        out_type=jax.ShapeDtypeStruct((STAGE_ROWS, 2 * DIM), jnp.float32),
        mesh=mesh,
        scratch_types=[
            pltpu.VMEM((R + L,), jnp.int32),
            pltpu.VMEM((R + L,), jnp.int32),
            pltpu.VMEM((DIM, BLK), jnp.float32),
            pltpu.VMEM((DIM, BLK), jnp.float32),
            pltpu.VMEM((SCB, 2 * DIM), jnp.float32),
            pltpu.VMEM((SCB,), jnp.int32),
            pltpu.VMEM((2 * L,), jnp.int32),
            pltpu.VMEM((2 * L,), jnp.int32),
            pltpu.VMEM((10 * L,), jnp.int32),
            pltpu.VMEM((10 * L,), jnp.int32),
            pltpu.VMEM((10 * L,), jnp.int32),
            pltpu.SemaphoreType.DMA,
            pltpu.SemaphoreType.DMA,
            pltpu.SemaphoreType.DMA,
        ],
        compiler_params=pltpu.CompilerParams(needs_layout_passes=False),
    )
    stage = k1(hidx, tidx, ehT, eh_tail)

    k2 = pl.kernel(
        _k2_body,
        out_type=jax.ShapeDtypeStruct((B,), jnp.float32),
        mesh=mesh,
        scratch_types=[
            pltpu.VMEM((BPW,), jnp.int32),
            pltpu.VMEM((BPW,), jnp.int32),
            pltpu.VMEM((BPW,), jnp.int32),
            pltpu.VMEM((BPW,), jnp.int32),
            pltpu.VMEM((SCB, 2 * DIM), jnp.float32),
            pltpu.VMEM((SCB, 2 * DIM), jnp.float32),
            pltpu.VMEM((SCB, 2 * DIM), jnp.float32),
            pltpu.VMEM((SCB,), jnp.float32),
            pltpu.VMEM((SCB,), jnp.float32),
            pltpu.VMEM((BPW,), jnp.float32),
            pltpu.SemaphoreType.DMA,
        ],
        compiler_params=pltpu.CompilerParams(needs_layout_passes=False),
    )
    return k2(stage, hidx, tidx, ridx, rvh2, bias0, bias1)


# speedup vs baseline: 1.6679x; 1.1366x over previous
import functools
import jax
import jax.numpy as jnp
from jax import lax
from jax.experimental import pallas as pl
from jax.experimental.pallas import tpu as pltpu
from jax.experimental.pallas import tpu_sc as plsc

N_ENT = 1000000
N_REL = 1000
DIM = 64
B = 16384

NC = 2
NS = 16
L = 16
NW = NC * NS
BPW = B // NW
R = 2 * B
BLK = 128
NBLK_FULL = 7812
TAIL_BASE = NBLK_FULL * BLK
IT_FULL = 244
NORD = 246
DUMP = R
STAGE_ROWS = R + 128
SCB = 128


def _k1_body(hidx, tidx, ehT, eh_tail, stage,
             idx_all, hits, horder, col0, col1, stage_b, slotbuf,
             cnt_v, off_v, cur_v,
             sem0, sem1, ssem):
    wid = lax.axis_index("s") * NC + lax.axis_index("c")
    lanes = lax.iota(jnp.int32, L)
    zero16 = jnp.full((L,), 0, jnp.int32)

    pltpu.sync_copy(hidx, idx_all.at[pl.ds(0, B)])

    def store1(ref, pos, val):
        plsc.store_scatter(ref, [zero16 + pos], zero16 + val,
                           mask=lanes == 0)

    def read1(ref, pos):
        return ref[pl.ds(pos, L)][0]

    def zcnt(i, carry):
        cnt_v[pl.ds(i * L, L)] = zero16
        return carry

    lax.fori_loop(0, (NORD + L - 1) // L + 1, zcnt, 0, unroll=False)

    def scan_for(slot_base):
        def scan(c, nh):
            e = idx_all[pl.ds(c * L, L)]
            slots = jnp.full((L,), slot_base + c * L, jnp.int32) + lanes
            m = lax.bitwise_and(lax.shift_right_logical(e, 7),
                                jnp.full((L,), NW - 1, jnp.int32)) == wid
            packed = lax.bitwise_or(
                lax.bitwise_or(
                    lax.shift_left(lax.shift_right_logical(e, 12),
                                   jnp.full((L,), 22, jnp.int32)),
                    lax.shift_left(
                        lax.bitwise_and(e, jnp.full((L,), BLK - 1,
                                                    jnp.int32)),
                        jnp.full((L,), 15, jnp.int32))),
                slots)
            plsc.store_compressed(hits.at[pl.ds(nh, L)], packed, mask=m)
            n = plsc.all_reduce_population_count(m)[0]
            return nh + n
        return scan

    nhits = lax.fori_loop(0, B // L, scan_for(0), 0, unroll=4)
    pltpu.sync_copy(tidx, idx_all.at[pl.ds(0, B)])
    nhits = lax.fori_loop(0, B // L, scan_for(B), nhits, unroll=4)

    def count(j, carry):
        o = lax.shift_right_logical(read1(hits, j), 22)
        store1(cnt_v, o, read1(cnt_v, o) + 1)
        return carry

    lax.fori_loop(0, nhits, count, 0, unroll=False)

    def prefix(i, s):
        store1(off_v, i, s)
        store1(cur_v, i, s)
        return s + read1(cnt_v, i)

    total = lax.fori_loop(0, NORD, prefix, 0, unroll=False)
    store1(off_v, NORD, total)

    def place(j, carry):
        v = read1(hits, j)
        o = lax.shift_right_logical(v, 22)
        p = read1(cur_v, o)
        store1(horder, p, v)
        store1(cur_v, o, p + 1)
        return carry

    lax.fori_loop(0, nhits, place, 0, unroll=False)

    def prefill(i, carry):
        slotbuf[pl.ds(i * L, L)] = jnp.full((L,), DUMP, jnp.int32)
        return carry

    lax.fori_loop(0, SCB // L, prefill, 0, unroll=False)

    def process_block(i, col, cnt):
        lo = read1(off_v, i)
        hi = read1(off_v, i + 1)

        def extract(j, cnt3):
            flushing = cnt3 == SCB

            @pl.when(flushing)
            def _():
                pltpu.async_copy(stage_b, stage.at[slotbuf], ssem).wait()
                lax.fori_loop(0, SCB // L, prefill, 0, unroll=False)

            cnt3 = jnp.where(flushing, 0, cnt3)
            v = read1(horder, j)
            slot = lax.bitwise_and(v, (1 << 15) - 1)
            el = lax.bitwise_and(lax.shift_right_logical(v, 15), BLK - 1)
            elv = zero16 + el
            rowv = zero16 + cnt3
            for k in range(DIM // L):
                dv = jnp.full((L,), k * L, jnp.int32) + lanes
                vv = plsc.load_gather(col, [dv, elv])
                plsc.store_scatter(stage_b, [rowv, dv], vv)
            store1(slotbuf, cnt3, slot)
            return cnt3 + 1

        return lax.fori_loop(lo, hi, extract, cnt, unroll=False)

    def pbase(i):
        return jnp.minimum((wid + NW * i) * BLK, (NBLK_FULL - 1) * BLK)

    pltpu.async_copy(ehT.at[:, pl.ds(pbase(0), BLK)], col0, sem0)
    pltpu.async_copy(ehT.at[:, pl.ds(pbase(1), BLK)], col1, sem1)

    def pair(p, cnt):
        pltpu.make_async_copy(ehT.at[:, pl.ds(pbase(2 * p), BLK)],
                              col0, sem0).wait()
        cnt = process_block(2 * p, col0, cnt)
        pltpu.async_copy(ehT.at[:, pl.ds(pbase(2 * p + 2), BLK)], col0, sem0)
        pltpu.make_async_copy(ehT.at[:, pl.ds(pbase(2 * p + 1), BLK)],
                              col1, sem1).wait()
        cnt = process_block(2 * p + 1, col1, cnt)
        pltpu.async_copy(ehT.at[:, pl.ds(pbase(2 * p + 3), BLK)], col1, sem1)
        return cnt

    cnt = lax.fori_loop(0, IT_FULL // 2, pair, 0, unroll=False)

    pltpu.make_async_copy(ehT.at[:, pl.ds(pbase(IT_FULL), BLK)],
                          col0, sem0).wait()
    pltpu.make_async_copy(ehT.at[:, pl.ds(pbase(IT_FULL + 1), BLK)],
                          col1, sem1).wait()

    @pl.when(wid < 4)
    def _():
        pltpu.sync_copy(ehT.at[:, pl.ds((wid + NW * IT_FULL) * BLK, BLK)],
                        col0)

    @pl.when(wid == 4)
    def _():
        pltpu.sync_copy(eh_tail, col0)

    cnt = process_block(IT_FULL, col0, cnt)

    @pl.when(cnt > 0)
    def _():
        pltpu.async_copy(stage_b, stage.at[slotbuf], ssem).wait()


def _k2_body(stage, hidx, tidx, ridx, rvh2, b0, b1, out,
             rel_v, rpair_v, hi_v, ti_v, h_r, t_r, rv_r, b0_v, b1_v, out_v,
             sem):
    wid = lax.axis_index("s") * NC + lax.axis_index("c")
    base = wid * BPW
    lanes = lax.iota(jnp.int32, L)

    pltpu.sync_copy(ridx.at[pl.ds(base, BPW)], rel_v)
    pltpu.sync_copy(hidx.at[pl.ds(base, BPW)], hi_v)
    pltpu.sync_copy(tidx.at[pl.ds(base, BPW)], ti_v)

    def pairs(i, carry):
        s = pl.ds(i * L, L)
        rpair_v[s] = lax.shift_right_logical(rel_v[s], 1)
        return carry

    lax.fori_loop(0, BPW // L, pairs, 0, unroll=False)

    NBAT = BPW // SCB

    def batch(bi, carry):
        s0 = base + bi * SCB
        c1 = pltpu.async_copy(stage.at[pl.ds(s0, SCB), :], h_r, sem)
        c2 = pltpu.async_copy(stage.at[pl.ds(B + s0, SCB), :], t_r, sem)
        c3 = pltpu.async_copy(rvh2.at[rpair_v.at[pl.ds(bi * SCB, SCB)]],
                              rv_r, sem)
        c4 = pltpu.async_copy(b0.at[hi_v.at[pl.ds(bi * SCB, SCB)]], b0_v, sem)
        c5 = pltpu.async_copy(b1.at[ti_v.at[pl.ds(bi * SCB, SCB)]], b1_v, sem)
        c1.wait(); c2.wait(); c3.wait(); c4.wait(); c5.wait()

        def group(g, carry2):
            req = jnp.full((L,), g * L, jnp.int32) + lanes
            rh = lax.bitwise_and(
                rel_v[pl.ds(bi * SCB + g * L, L)],
                jnp.full((L,), 1, jnp.int32)) * DIM
            acc = jnp.zeros((L,), jnp.float32)
            for d in range(DIM):
                col = jnp.full((L,), d, jnp.int32)
                hv = plsc.load_gather(h_r, [req, col])
                tv = plsc.load_gather(t_r, [req, col])
                rv = plsc.load_gather(rv_r, [req, rh + col])
                diff = hv - tv - rv
                acc = acc + diff * diff
            gs = pl.ds(g * L, L)
            out_v[pl.ds(bi * SCB + g * L, L)] = b0_v[gs] + b1_v[gs] - acc
            return carry2

        lax.fori_loop(0, SCB // L, group, 0, unroll=False)
        return carry

    lax.fori_loop(0, NBAT, batch, 0, unroll=False)

    pltpu.sync_copy(out_v, out.at[pl.ds(base, BPW)])


@functools.partial(jax.jit, static_argnames=())
def kernel(head_idx, rel1_idx, tail_idx, rel2_idx, Eh, rvh, bias0, bias1):
    del rel2_idx
    hidx = head_idx.astype(jnp.int32)
    tidx = tail_idx.astype(jnp.int32)
    ridx = rel1_idx.astype(jnp.int32)
    ehT = Eh.T
    eh_tail = jnp.pad(Eh[TAIL_BASE:, :].T, ((0, 0), (0, DIM)))
    rvh2 = rvh.reshape(N_REL // 2, 2 * DIM)
    mesh = plsc.VectorSubcoreMesh(core_axis_name="c", subcore_axis_name="s")

    k1 = pl.kernel(
        _k1_body,
        out_type=jax.ShapeDtypeStruct((STAGE_ROWS, 2 * DIM), jnp.float32),
        mesh=mesh,
        scratch_types=[
            pltpu.VMEM((B + L,), jnp.int32),
            pltpu.VMEM((R + L,), jnp.int32),
            pltpu.VMEM((R + L,), jnp.int32),
            pltpu.VMEM((DIM, BLK), jnp.float32),
            pltpu.VMEM((DIM, BLK), jnp.float32),
            pltpu.VMEM((SCB, 2 * DIM), jnp.float32),
            pltpu.VMEM((SCB,), jnp.int32),
            pltpu.VMEM((18 * L,), jnp.int32),
            pltpu.VMEM((18 * L,), jnp.int32),
            pltpu.VMEM((18 * L,), jnp.int32),
            pltpu.SemaphoreType.DMA,
            pltpu.SemaphoreType.DMA,
            pltpu.SemaphoreType.DMA,
        ],
        compiler_params=pltpu.CompilerParams(needs_layout_passes=False),
    )
    stage = k1(hidx, tidx, ehT, eh_tail)

    k2 = pl.kernel(
        _k2_body,
        out_type=jax.ShapeDtypeStruct((B,), jnp.float32),
        mesh=mesh,
        scratch_types=[
            pltpu.VMEM((BPW,), jnp.int32),
            pltpu.VMEM((BPW,), jnp.int32),
            pltpu.VMEM((BPW,), jnp.int32),
            pltpu.VMEM((BPW,), jnp.int32),
            pltpu.VMEM((SCB, 2 * DIM), jnp.float32),
            pltpu.VMEM((SCB, 2 * DIM), jnp.float32),
            pltpu.VMEM((SCB, 2 * DIM), jnp.float32),
            pltpu.VMEM((SCB,), jnp.float32),
            pltpu.VMEM((SCB,), jnp.float32),
            pltpu.VMEM((BPW,), jnp.float32),
            pltpu.SemaphoreType.DMA,
        ],
        compiler_params=pltpu.CompilerParams(needs_layout_passes=False),
    )
    return k2(stage, hidx, tidx, ridx, rvh2, bias0, bias1)


# speedup vs baseline: 2.1505x; 1.2893x over previous
import functools
import jax
import jax.numpy as jnp
from jax import lax
from jax.experimental import pallas as pl
from jax.experimental.pallas import tpu as pltpu
from jax.experimental.pallas import tpu_sc as plsc

N_ENT = 1000000
N_REL = 1000
DIM = 64
B = 16384

NC = 2
NS = 16
L = 16
NW = NC * NS
BPW = B // NW
R = 2 * B
BLK = 128
NBLK_FULL = 7812
TAIL_BASE = NBLK_FULL * BLK
IT_FULL = 244
NORD = 246
DUMP = R
STAGE_ROWS = R + 128
SCB = 64
NRING = 4


def _k1_body(hidx, tidx, ehT, eh_tail, stage,
             idx_all, hits, horder, col0, col1, col2, col3, stage_b, slotbuf,
             cnt_v, off_v, cur_v,
             sem0, sem1, sem2, sem3, ssem):
    wid = lax.axis_index("s") * NC + lax.axis_index("c")
    lanes = lax.iota(jnp.int32, L)
    zero16 = jnp.full((L,), 0, jnp.int32)

    pltpu.sync_copy(hidx, idx_all.at[pl.ds(0, B)])

    def store1(ref, pos, val):
        plsc.store_scatter(ref, [zero16 + pos], zero16 + val,
                           mask=lanes == 0)

    def read1(ref, pos):
        return ref[pl.ds(pos, L)][0]

    def zcnt(i, carry):
        cnt_v[pl.ds(i * L, L)] = zero16
        return carry

    lax.fori_loop(0, (NORD + L - 1) // L + 1, zcnt, 0, unroll=False)

    def scan_for(slot_base):
        def scan(c, nh):
            e = idx_all[pl.ds(c * L, L)]
            slots = jnp.full((L,), slot_base + c * L, jnp.int32) + lanes
            m = lax.bitwise_and(lax.shift_right_logical(e, 7),
                                jnp.full((L,), NW - 1, jnp.int32)) == wid
            packed = lax.bitwise_or(
                lax.bitwise_or(
                    lax.shift_left(lax.shift_right_logical(e, 12),
                                   jnp.full((L,), 22, jnp.int32)),
                    lax.shift_left(
                        lax.bitwise_and(e, jnp.full((L,), BLK - 1,
                                                    jnp.int32)),
                        jnp.full((L,), 15, jnp.int32))),
                slots)
            plsc.store_compressed(hits.at[pl.ds(nh, L)], packed, mask=m)
            n = plsc.all_reduce_population_count(m)[0]
            return nh + n
        return scan

    nhits = lax.fori_loop(0, B // L, scan_for(0), 0, unroll=4)
    pltpu.sync_copy(tidx, idx_all.at[pl.ds(0, B)])
    nhits = lax.fori_loop(0, B // L, scan_for(B), nhits, unroll=4)

    def count(j, carry):
        o = lax.shift_right_logical(read1(hits, j), 22)
        store1(cnt_v, o, read1(cnt_v, o) + 1)
        return carry

    lax.fori_loop(0, nhits, count, 0, unroll=False)

    def prefix(i, s):
        store1(off_v, i, s)
        store1(cur_v, i, s)
        return s + read1(cnt_v, i)

    total = lax.fori_loop(0, NORD, prefix, 0, unroll=False)
    store1(off_v, NORD, total)

    def place(j, carry):
        v = read1(hits, j)
        o = lax.shift_right_logical(v, 22)
        p = read1(cur_v, o)
        store1(horder, p, v)
        store1(cur_v, o, p + 1)
        return carry

    lax.fori_loop(0, nhits, place, 0, unroll=False)

    def prefill(i, carry):
        slotbuf[pl.ds(i * L, L)] = jnp.full((L,), DUMP, jnp.int32)
        return carry

    lax.fori_loop(0, SCB // L, prefill, 0, unroll=False)

    def process_block(i, col, cnt):
        lo = read1(off_v, i)
        hi = read1(off_v, i + 1)

        def extract(j, cnt3):
            flushing = cnt3 == SCB

            @pl.when(flushing)
            def _():
                pltpu.async_copy(stage_b, stage.at[slotbuf], ssem).wait()
                lax.fori_loop(0, SCB // L, prefill, 0, unroll=False)

            cnt3 = jnp.where(flushing, 0, cnt3)
            v = read1(horder, j)
            slot = lax.bitwise_and(v, (1 << 15) - 1)
            el = lax.bitwise_and(lax.shift_right_logical(v, 15), BLK - 1)
            elv = zero16 + el
            rowv = zero16 + cnt3
            for k in range(DIM // L):
                dv = jnp.full((L,), k * L, jnp.int32) + lanes
                vv = plsc.load_gather(col, [dv, elv])
                plsc.store_scatter(stage_b, [rowv, dv], vv)
            store1(slotbuf, cnt3, slot)
            return cnt3 + 1

        return lax.fori_loop(lo, hi, extract, cnt, unroll=False)

    def pbase(i):
        return jnp.minimum((wid + NW * i) * BLK, (NBLK_FULL - 1) * BLK)

    cols = [col0, col1, col2, col3]
    sems = [sem0, sem1, sem2, sem3]
    for k in range(NRING):
        pltpu.async_copy(ehT.at[:, pl.ds(pbase(k), BLK)], cols[k], sems[k])

    def ring(g, cnt):
        for k in range(NRING):
            i = NRING * g + k
            pltpu.make_async_copy(ehT.at[:, pl.ds(pbase(i), BLK)],
                                  cols[k], sems[k]).wait()
            cnt = process_block(i, cols[k], cnt)
            pltpu.async_copy(ehT.at[:, pl.ds(pbase(i + NRING), BLK)],
                             cols[k], sems[k])
        return cnt

    cnt = lax.fori_loop(0, IT_FULL // NRING, ring, 0, unroll=False)

    for k in range(NRING):
        pltpu.make_async_copy(ehT.at[:, pl.ds(pbase(IT_FULL + k), BLK)],
                              cols[k], sems[k]).wait()

    @pl.when(wid < 4)
    def _():
        pltpu.sync_copy(ehT.at[:, pl.ds((wid + NW * IT_FULL) * BLK, BLK)],
                        col0)

    @pl.when(wid == 4)
    def _():
        pltpu.sync_copy(eh_tail, col0)

    cnt = process_block(IT_FULL, col0, cnt)

    @pl.when(cnt > 0)
    def _():
        pltpu.async_copy(stage_b, stage.at[slotbuf], ssem).wait()


def _k2_body(stage, hidx, tidx, ridx, rvh2, b0, b1, out,
             rel_v, rpair_v, hi_v, ti_v, h_r, t_r, rv_r, b0_v, b1_v, out_v,
             sem):
    wid = lax.axis_index("s") * NC + lax.axis_index("c")
    base = wid * BPW
    lanes = lax.iota(jnp.int32, L)

    pltpu.sync_copy(ridx.at[pl.ds(base, BPW)], rel_v)
    pltpu.sync_copy(hidx.at[pl.ds(base, BPW)], hi_v)
    pltpu.sync_copy(tidx.at[pl.ds(base, BPW)], ti_v)

    def pairs(i, carry):
        s = pl.ds(i * L, L)
        rpair_v[s] = lax.shift_right_logical(rel_v[s], 1)
        return carry

    lax.fori_loop(0, BPW // L, pairs, 0, unroll=False)

    NBAT = BPW // SCB

    def batch(bi, carry):
        s0 = base + bi * SCB
        c1 = pltpu.async_copy(stage.at[pl.ds(s0, SCB), :], h_r, sem)
        c2 = pltpu.async_copy(stage.at[pl.ds(B + s0, SCB), :], t_r, sem)
        c3 = pltpu.async_copy(rvh2.at[rpair_v.at[pl.ds(bi * SCB, SCB)]],
                              rv_r, sem)
        c4 = pltpu.async_copy(b0.at[hi_v.at[pl.ds(bi * SCB, SCB)]], b0_v, sem)
        c5 = pltpu.async_copy(b1.at[ti_v.at[pl.ds(bi * SCB, SCB)]], b1_v, sem)
        c1.wait(); c2.wait(); c3.wait(); c4.wait(); c5.wait()

        def group(g, carry2):
            req = jnp.full((L,), g * L, jnp.int32) + lanes
            rh = lax.bitwise_and(
                rel_v[pl.ds(bi * SCB + g * L, L)],
                jnp.full((L,), 1, jnp.int32)) * DIM
            acc = jnp.zeros((L,), jnp.float32)
            for d in range(DIM):
                col = jnp.full((L,), d, jnp.int32)
                hv = plsc.load_gather(h_r, [req, col])
                tv = plsc.load_gather(t_r, [req, col])
                rv = plsc.load_gather(rv_r, [req, rh + col])
                diff = hv - tv - rv
                acc = acc + diff * diff
            gs = pl.ds(g * L, L)
            out_v[pl.ds(bi * SCB + g * L, L)] = b0_v[gs] + b1_v[gs] - acc
            return carry2

        lax.fori_loop(0, SCB // L, group, 0, unroll=False)
        return carry

    lax.fori_loop(0, NBAT, batch, 0, unroll=False)

    pltpu.sync_copy(out_v, out.at[pl.ds(base, BPW)])


@functools.partial(jax.jit, static_argnames=())
def kernel(head_idx, rel1_idx, tail_idx, rel2_idx, Eh, rvh, bias0, bias1):
    del rel2_idx
    hidx = head_idx.astype(jnp.int32)
    tidx = tail_idx.astype(jnp.int32)
    ridx = rel1_idx.astype(jnp.int32)
    ehT = Eh.T
    eh_tail = jnp.pad(Eh[TAIL_BASE:, :].T, ((0, 0), (0, DIM)))
    rvh2 = rvh.reshape(N_REL // 2, 2 * DIM)
    mesh = plsc.VectorSubcoreMesh(core_axis_name="c", subcore_axis_name="s")

    k1 = pl.kernel(
        _k1_body,
        out_type=jax.ShapeDtypeStruct((STAGE_ROWS, 2 * DIM), jnp.float32),
        mesh=mesh,
        scratch_types=[
            pltpu.VMEM((B + L,), jnp.int32),
            pltpu.VMEM((R + L,), jnp.int32),
            pltpu.VMEM((R + L,), jnp.int32),
            pltpu.VMEM((DIM, BLK), jnp.float32),
            pltpu.VMEM((DIM, BLK), jnp.float32),
            pltpu.VMEM((DIM, BLK), jnp.float32),
            pltpu.VMEM((DIM, BLK), jnp.float32),
            pltpu.VMEM((SCB, 2 * DIM), jnp.float32),
            pltpu.VMEM((SCB,), jnp.int32),
            pltpu.VMEM((18 * L,), jnp.int32),
            pltpu.VMEM((18 * L,), jnp.int32),
            pltpu.VMEM((18 * L,), jnp.int32),
            pltpu.SemaphoreType.DMA,
            pltpu.SemaphoreType.DMA,
            pltpu.SemaphoreType.DMA,
            pltpu.SemaphoreType.DMA,
            pltpu.SemaphoreType.DMA,
        ],
        compiler_params=pltpu.CompilerParams(needs_layout_passes=False),
    )
    stage = k1(hidx, tidx, ehT, eh_tail)

    k2 = pl.kernel(
        _k2_body,
        out_type=jax.ShapeDtypeStruct((B,), jnp.float32),
        mesh=mesh,
        scratch_types=[
            pltpu.VMEM((BPW,), jnp.int32),
            pltpu.VMEM((BPW,), jnp.int32),
            pltpu.VMEM((BPW,), jnp.int32),
            pltpu.VMEM((BPW,), jnp.int32),
            pltpu.VMEM((SCB, 2 * DIM), jnp.float32),
            pltpu.VMEM((SCB, 2 * DIM), jnp.float32),
            pltpu.VMEM((SCB, 2 * DIM), jnp.float32),
            pltpu.VMEM((SCB,), jnp.float32),
            pltpu.VMEM((SCB,), jnp.float32),
            pltpu.VMEM((BPW,), jnp.float32),
            pltpu.SemaphoreType.DMA,
        ],
        compiler_params=pltpu.CompilerParams(needs_layout_passes=False),
    )
    return k2(stage, hidx, tidx, ridx, rvh2, bias0, bias1)


# speedup vs baseline: 2.1991x; 1.0226x over previous
import functools
import jax
import jax.numpy as jnp
from jax import lax
from jax.experimental import pallas as pl
from jax.experimental.pallas import tpu as pltpu
from jax.experimental.pallas import tpu_sc as plsc

N_ENT = 1000000
N_REL = 1000
DIM = 64
B = 16384

NC = 2
NS = 16
L = 16
NW = NC * NS
BPW = B // NW
R = 2 * B
BLK = 128
NBLK_FULL = 7812
TAIL_BASE = NBLK_FULL * BLK
IT_FULL = 244
NORD = 246
DUMP = R
STAGE_ROWS = R + 128
SCB = 64
NRING = 4


def _k1_body(hidx, tidx, ehT, eh_tail, stage,
             idx_all, hits, horder, col0, col1, col2, col3, stage_b, slotbuf,
             cnt_v, off_v, cur_v,
             sem0, sem1, sem2, sem3, ssem):
    wid = lax.axis_index("s") * NC + lax.axis_index("c")
    lanes = lax.iota(jnp.int32, L)
    zero16 = jnp.full((L,), 0, jnp.int32)

    pltpu.sync_copy(hidx, idx_all.at[pl.ds(0, B)])

    def store1(ref, pos, val):
        plsc.store_scatter(ref, [zero16 + pos], zero16 + val,
                           mask=lanes == 0)

    def read1(ref, pos):
        return ref[pl.ds(pos, L)][0]

    def zcnt(i, carry):
        cnt_v[pl.ds(i * L, L)] = zero16
        return carry

    lax.fori_loop(0, (NORD + L - 1) // L + 1, zcnt, 0, unroll=False)

    def scan_for(slot_base):
        def scan(c, nh):
            e = idx_all[pl.ds(c * L, L)]
            slots = jnp.full((L,), slot_base + c * L, jnp.int32) + lanes
            m = lax.bitwise_and(lax.shift_right_logical(e, 7),
                                jnp.full((L,), NW - 1, jnp.int32)) == wid
            packed = lax.bitwise_or(
                lax.bitwise_or(
                    lax.shift_left(lax.shift_right_logical(e, 12),
                                   jnp.full((L,), 22, jnp.int32)),
                    lax.shift_left(
                        lax.bitwise_and(e, jnp.full((L,), BLK - 1,
                                                    jnp.int32)),
                        jnp.full((L,), 15, jnp.int32))),
                slots)
            plsc.store_compressed(hits.at[pl.ds(nh, L)], packed, mask=m)
            n = plsc.all_reduce_population_count(m)[0]
            return nh + n
        return scan

    nhits = lax.fori_loop(0, B // L, scan_for(0), 0, unroll=4)
    pltpu.sync_copy(tidx, idx_all.at[pl.ds(0, B)])
    nhits = lax.fori_loop(0, B // L, scan_for(B), nhits, unroll=4)

    def count(j, carry):
        o = lax.shift_right_logical(read1(hits, j), 22)
        store1(cnt_v, o, read1(cnt_v, o) + 1)
        return carry

    lax.fori_loop(0, nhits, count, 0, unroll=False)

    def prefix(i, s):
        store1(off_v, i, s)
        store1(cur_v, i, s)
        return s + read1(cnt_v, i)

    total = lax.fori_loop(0, NORD, prefix, 0, unroll=False)
    store1(off_v, NORD, total)

    def place(j, carry):
        v = read1(hits, j)
        o = lax.shift_right_logical(v, 22)
        p = read1(cur_v, o)
        store1(horder, p, v)
        store1(cur_v, o, p + 1)
        return carry

    lax.fori_loop(0, nhits, place, 0, unroll=False)

    def prefill(i, carry):
        slotbuf[pl.ds(i * L, L)] = jnp.full((L,), DUMP, jnp.int32)
        return carry

    lax.fori_loop(0, SCB // L, prefill, 0, unroll=False)

    def process_block(i, col, cnt):
        lo = read1(off_v, i)
        hi = read1(off_v, i + 1)

        def extract(j, cnt3):
            flushing = cnt3 == SCB

            @pl.when(flushing)
            def _():
                pltpu.async_copy(stage_b, stage.at[slotbuf], ssem).wait()
                lax.fori_loop(0, SCB // L, prefill, 0, unroll=False)

            cnt3 = jnp.where(flushing, 0, cnt3)
            v = read1(horder, j)
            slot = lax.bitwise_and(v, (1 << 15) - 1)
            el = lax.bitwise_and(lax.shift_right_logical(v, 15), BLK - 1)
            elv = zero16 + el
            rowv = zero16 + cnt3
            for k in range(DIM // L):
                dv = jnp.full((L,), k * L, jnp.int32) + lanes
                vv = plsc.load_gather(col, [dv, elv])
                plsc.store_scatter(stage_b, [rowv, dv], vv)
            store1(slotbuf, cnt3, slot)
            return cnt3 + 1

        return lax.fori_loop(lo, hi, extract, cnt, unroll=False)

    def pbase(i):
        return jnp.minimum((wid + NW * i) * BLK, (NBLK_FULL - 1) * BLK)

    cols = [col0, col1, col2, col3]
    sems = [sem0, sem1, sem2, sem3]
    for k in range(NRING):
        pltpu.async_copy(ehT.at[:, pl.ds(pbase(k), BLK)], cols[k], sems[k])

    def ring(g, cnt):
        for k in range(NRING):
            i = NRING * g + k
            pltpu.make_async_copy(ehT.at[:, pl.ds(pbase(i), BLK)],
                                  cols[k], sems[k]).wait()
            cnt = process_block(i, cols[k], cnt)
            pltpu.async_copy(ehT.at[:, pl.ds(pbase(i + NRING), BLK)],
                             cols[k], sems[k])
        return cnt

    cnt = lax.fori_loop(0, IT_FULL // NRING, ring, 0, unroll=False)

    for k in range(NRING):
        pltpu.make_async_copy(ehT.at[:, pl.ds(pbase(IT_FULL + k), BLK)],
                              cols[k], sems[k]).wait()

    @pl.when(wid < 4)
    def _():
        pltpu.sync_copy(ehT.at[:, pl.ds((wid + NW * IT_FULL) * BLK, BLK)],
                        col0)

    @pl.when(wid == 4)
    def _():
        pltpu.sync_copy(eh_tail, col0)

    cnt = process_block(IT_FULL, col0, cnt)

    @pl.when(cnt > 0)
    def _():
        pltpu.async_copy(stage_b, stage.at[slotbuf], ssem).wait()


def _k2_body(stage, hidx, tidx, ridx, rvh2, b0, b1, out,
             rel_v, rpair_v, hi_v, ti_v,
             h_r0, t_r0, rv_r0, b0_v0, b1_v0,
             h_r1, t_r1, rv_r1, b0_v1, b1_v1,
             out_v, semA, semB):
    wid = lax.axis_index("s") * NC + lax.axis_index("c")
    base = wid * BPW
    lanes = lax.iota(jnp.int32, L)

    pltpu.sync_copy(ridx.at[pl.ds(base, BPW)], rel_v)
    pltpu.sync_copy(hidx.at[pl.ds(base, BPW)], hi_v)
    pltpu.sync_copy(tidx.at[pl.ds(base, BPW)], ti_v)

    def pairs(i, carry):
        s = pl.ds(i * L, L)
        rpair_v[s] = lax.shift_right_logical(rel_v[s], 1)
        return carry

    lax.fori_loop(0, BPW // L, pairs, 0, unroll=False)

    KB = 128
    NBAT = BPW // KB
    sets = [(h_r0, t_r0, rv_r0, b0_v0, b1_v0, semA),
            (h_r1, t_r1, rv_r1, b0_v1, b1_v1, semB)]

    def fire(bi, st):
        h_r, t_r, rv_r, b0_v, b1_v, sem = st
        s0 = base + bi * KB
        return [
            pltpu.async_copy(stage.at[pl.ds(s0, KB), :], h_r, sem),
            pltpu.async_copy(stage.at[pl.ds(B + s0, KB), :], t_r, sem),
            pltpu.async_copy(rvh2.at[rpair_v.at[pl.ds(bi * KB, KB)]],
                             rv_r, sem),
            pltpu.async_copy(b0.at[hi_v.at[pl.ds(bi * KB, KB)]], b0_v, sem),
            pltpu.async_copy(b1.at[ti_v.at[pl.ds(bi * KB, KB)]], b1_v, sem),
        ]

    def compute(bi, st):
        h_r, t_r, rv_r, b0_v, b1_v, _ = st

        def group(g, carry2):
            req = jnp.full((L,), g * L, jnp.int32) + lanes
            rh = lax.bitwise_and(
                rel_v[pl.ds(bi * KB + g * L, L)],
                jnp.full((L,), 1, jnp.int32)) * DIM
            acc = jnp.zeros((L,), jnp.float32)
            for d in range(DIM):
                col = jnp.full((L,), d, jnp.int32)
                hv = plsc.load_gather(h_r, [req, col])
                tv = plsc.load_gather(t_r, [req, col])
                rv = plsc.load_gather(rv_r, [req, rh + col])
                diff = hv - tv - rv
                acc = acc + diff * diff
            gs = pl.ds(g * L, L)
            out_v[pl.ds(bi * KB + g * L, L)] = b0_v[gs] + b1_v[gs] - acc
            return carry2

        lax.fori_loop(0, KB // L, group, 0, unroll=False)

    pending = fire(0, sets[0])
    for bi in range(NBAT):
        st = sets[bi % 2]
        for c in pending:
            c.wait()
        if bi + 1 < NBAT:
            nxt = fire(bi + 1, sets[(bi + 1) % 2])
        else:
            nxt = []
        compute(bi, st)
        pending = nxt

    pltpu.sync_copy(out_v, out.at[pl.ds(base, BPW)])


@functools.partial(jax.jit, static_argnames=())
def kernel(head_idx, rel1_idx, tail_idx, rel2_idx, Eh, rvh, bias0, bias1):
    del rel2_idx
    hidx = head_idx.astype(jnp.int32)
    tidx = tail_idx.astype(jnp.int32)
    ridx = rel1_idx.astype(jnp.int32)
    ehT = Eh.T
    eh_tail = jnp.pad(Eh[TAIL_BASE:, :].T, ((0, 0), (0, DIM)))
    rvh2 = rvh.reshape(N_REL // 2, 2 * DIM)
    mesh = plsc.VectorSubcoreMesh(core_axis_name="c", subcore_axis_name="s")

    k1 = pl.kernel(
        _k1_body,
        out_type=jax.ShapeDtypeStruct((STAGE_ROWS, 2 * DIM), jnp.float32),
        mesh=mesh,
        scratch_types=[
            pltpu.VMEM((B + L,), jnp.int32),
            pltpu.VMEM((R + L,), jnp.int32),
            pltpu.VMEM((R + L,), jnp.int32),
            pltpu.VMEM((DIM, BLK), jnp.float32),
            pltpu.VMEM((DIM, BLK), jnp.float32),
            pltpu.VMEM((DIM, BLK), jnp.float32),
            pltpu.VMEM((DIM, BLK), jnp.float32),
            pltpu.VMEM((SCB, 2 * DIM), jnp.float32),
            pltpu.VMEM((SCB,), jnp.int32),
            pltpu.VMEM((18 * L,), jnp.int32),
            pltpu.VMEM((18 * L,), jnp.int32),
            pltpu.VMEM((18 * L,), jnp.int32),
            pltpu.SemaphoreType.DMA,
            pltpu.SemaphoreType.DMA,
            pltpu.SemaphoreType.DMA,
            pltpu.SemaphoreType.DMA,
            pltpu.SemaphoreType.DMA,
        ],
        compiler_params=pltpu.CompilerParams(needs_layout_passes=False),
    )
    stage = k1(hidx, tidx, ehT, eh_tail)

    k2 = pl.kernel(
        _k2_body,
        out_type=jax.ShapeDtypeStruct((B,), jnp.float32),
        mesh=mesh,
        scratch_types=[
            pltpu.VMEM((BPW,), jnp.int32),
            pltpu.VMEM((BPW,), jnp.int32),
            pltpu.VMEM((BPW,), jnp.int32),
            pltpu.VMEM((BPW,), jnp.int32),
            pltpu.VMEM((128, 2 * DIM), jnp.float32),
            pltpu.VMEM((128, 2 * DIM), jnp.float32),
            pltpu.VMEM((128, 2 * DIM), jnp.float32),
            pltpu.VMEM((128,), jnp.float32),
            pltpu.VMEM((128,), jnp.float32),
            pltpu.VMEM((128, 2 * DIM), jnp.float32),
            pltpu.VMEM((128, 2 * DIM), jnp.float32),
            pltpu.VMEM((128, 2 * DIM), jnp.float32),
            pltpu.VMEM((128,), jnp.float32),
            pltpu.VMEM((128,), jnp.float32),
            pltpu.VMEM((BPW,), jnp.float32),
            pltpu.SemaphoreType.DMA,
            pltpu.SemaphoreType.DMA,
        ],
        compiler_params=pltpu.CompilerParams(needs_layout_passes=False),
    )
    return k2(stage, hidx, tidx, ridx, rvh2, bias0, bias1)
